# baseline (speedup 1.0000x reference)
import functools

import jax
import jax.numpy as jnp
import numpy as np
from jax import lax
from jax.experimental import pallas as pl
from jax.experimental.pallas import tpu as pltpu
from jax.experimental.pallas import tpu_sc as plsc

B = 16384
C = 128
T = 1000
EPS = 0.0001

_NC = 2
_NS = 16
_NW = _NC * _NS
_BPW = B // _NW
_TPS = 64
_L = 16
_NG = _TPS // _L

_COS_COEFFS = (1.0, -19.73920440673828, 64.93910217285156,
               -85.44971466064453, 60.163578033447266,
               -25.950340270996094, 6.501813888549805)
_INV2PI = np.float32(1.0 / (2.0 * np.pi))
_RB = np.float32(1.5 * 2.0 ** 23)
_SQRT2 = np.float32(np.sqrt(2.0))
_EPS_SCALED = np.float32(np.sqrt(C) * EPS)


def _lane(v, j):
    return jnp.take_along_axis(v, jnp.full((_L,), j, jnp.int32), axis=0)


def _cos_turns(rf):
    k = (rf + _RB) - _RB
    t = rf - k
    s = t * t
    p = jnp.full((_L,), _COS_COEFFS[-1], jnp.float32)
    for c in _COS_COEFFS[-2::-1]:
        p = p * s + np.float32(c)
    return p


def _sc_body(t_hbm, loss_hbm, ab_hbm, lam_hbm, iw_hbm, fq_hbm, ph_hbm,
             w_hbm, mean_hbm, std_hbm, out_hbm, scaled_hbm,
             idx_v, loss_v, f1g_v, f2g_v, o1_v, o2_v, fq_v, ph_v, w_v,
             ms_v, ab_v, lam_v, iw_v, f1loc, f2loc, f1_sh, f2_sh,
             *sems):
    cid = lax.axis_index("c")
    sid = lax.axis_index("s")
    wid = sid * _NC + cid
    base = wid * _BPW
    tbase = jnp.minimum(sid * _TPS, T - _TPS)

    copies = [
        pltpu.async_copy(fq_hbm, fq_v, sems[0]),
        pltpu.async_copy(ph_hbm, ph_v, sems[1]),
        pltpu.async_copy(w_hbm, w_v, sems[2]),
        pltpu.async_copy(mean_hbm, ms_v.at[pl.ds(0, 1)], sems[3]),
        pltpu.async_copy(std_hbm, ms_v.at[pl.ds(8, 1)], sems[4]),
        pltpu.async_copy(ab_hbm.at[pl.ds(tbase, _TPS)], ab_v, sems[5]),
        pltpu.async_copy(lam_hbm.at[pl.ds(tbase, _TPS)], lam_v, sems[6]),
        pltpu.async_copy(iw_hbm.at[pl.ds(tbase, _TPS)], iw_v, sems[7]),
        pltpu.async_copy(t_hbm.at[pl.ds(base, _BPW)], idx_v, sems[8]),
        pltpu.async_copy(loss_hbm.at[pl.ds(base, _BPW)], loss_v, sems[9]),
    ]
    copies[2].wait()

    acc = jnp.zeros((_L,), jnp.float32)
    for v in range(C // _L):
        wv = w_v[pl.ds(v * _L, _L)]
        acc = acc + wv * wv
    svec = jnp.zeros((_L,), jnp.float32) + jnp.sum(acc)
    seed = jnp.full((_L,), np.int32(0x5F3759DF), jnp.int32) - \
        lax.shift_right_logical(plsc.bitcast(svec, jnp.int32), 1)
    y = plsc.bitcast(seed, jnp.float32)
    for _ in range(3):
        y = y * (np.float32(1.5) - (np.float32(0.5) * svec) * y * y)
    normv = svec * y
    scalev = _SQRT2 / (_EPS_SCALED + normv)

    copies[3].wait()
    copies[4].wait()
    copies[5].wait()
    msv = ms_v[pl.ds(0, _L)]
    meanv = _lane(msv, 0)
    stdv = _lane(msv, 8)
    cns = [(ab_v[pl.ds(g * _L, _L)] - meanv) / stdv for g in range(_NG)]

    copies[0].wait()
    copies[1].wait()
    zero = jnp.zeros((_L,), jnp.float32)
    accs = [zero] * _NG
    for c16 in range(C // _L):
        coff = c16 * _L
        ft = fq_v[pl.ds(coff, _L)] * _INV2PI
        pt = ph_v[pl.ds(coff, _L)] * _INV2PI
        w16 = w_v[pl.ds(coff, _L)]
        for j in range(_L):
            fb = _lane(ft, j)
            pb = _lane(pt, j)
            wb = _lane(w16, j)
            for g in range(_NG):
                accs[g] = accs[g] + _cos_turns(cns[g] * fb + pb) * wb

    copies[6].wait()
    copies[7].wait()
    for g in range(_NG):
        alw = accs[g] * scalev
        sl = pl.ds(g * _L, _L)
        f1loc[sl] = lam_v[sl] * jnp.exp(-alw)
        f2loc[sl] = iw_v[sl] * alw

    pltpu.sync_copy(f1loc, f1_sh.at[pl.ds(tbase, _TPS)])
    pltpu.sync_copy(f2loc, f2_sh.at[pl.ds(tbase, _TPS)])
    copies[8].wait()
    plsc.subcore_barrier()

    gathers = [
        pltpu.async_copy(f1_sh.at[idx_v], f1g_v, sems[10]),
        pltpu.async_copy(f2_sh.at[idx_v], f2g_v, sems[11]),
    ]
    copies[9].wait()
    for g_ in gathers:
        g_.wait()
    for i in range(_BPW // _L):
        sl = pl.ds(i * _L, _L)
        ls = loss_v[sl] * f1g_v[sl]
        o2_v[sl] = ls
        o1_v[sl] = ls + f2g_v[sl]
    pltpu.sync_copy(o1_v, out_hbm.at[pl.ds(base, _BPW)])
    pltpu.sync_copy(o2_v, scaled_hbm.at[pl.ds(base, _BPW)])


@functools.cache
def _get_sc_kernel():
    return pl.kernel(
        _sc_body,
        out_type=(jax.ShapeDtypeStruct((B,), jnp.float32),
                  jax.ShapeDtypeStruct((B,), jnp.float32)),
        mesh=plsc.VectorSubcoreMesh(core_axis_name="c",
                                    subcore_axis_name="s",
                                    num_cores=_NC, num_subcores=_NS),
        compiler_params=pltpu.CompilerParams(needs_layout_passes=False),
        scratch_types=[
            pltpu.VMEM((_BPW,), jnp.int32),
            pltpu.VMEM((_BPW,), jnp.float32),
            pltpu.VMEM((_BPW,), jnp.float32),
            pltpu.VMEM((_BPW,), jnp.float32),
            pltpu.VMEM((_BPW,), jnp.float32),
            pltpu.VMEM((_BPW,), jnp.float32),
            pltpu.VMEM((C,), jnp.float32),
            pltpu.VMEM((C,), jnp.float32),
            pltpu.VMEM((C,), jnp.float32),
            pltpu.VMEM((_L,), jnp.float32),
            pltpu.VMEM((_TPS,), jnp.float32),
            pltpu.VMEM((_TPS,), jnp.float32),
            pltpu.VMEM((_TPS,), jnp.float32),
            pltpu.VMEM((_TPS,), jnp.float32),
            pltpu.VMEM((_TPS,), jnp.float32),
            pltpu.VMEM_SHARED((T,), jnp.float32),
            pltpu.VMEM_SHARED((T,), jnp.float32),
        ] + [pltpu.SemaphoreType.DMA] * 18,
    )


def kernel(loss, timesteps, freqs, phases, weight, alphas_cumprod,
           a_bar_mean, a_bar_std, lambda_weights, importance_weights):
    loss_out, loss_scaled = _get_sc_kernel()(
        timesteps.astype(jnp.int32), loss, alphas_cumprod, lambda_weights,
        importance_weights, freqs, phases, weight.reshape(C),
        a_bar_mean.reshape(1), a_bar_std.reshape(1))
    return (loss_out, loss_scaled)


# speedup vs baseline: 1.3143x; 1.3143x over previous
import functools

import jax
import jax.numpy as jnp
import numpy as np
from jax import lax
from jax.experimental import pallas as pl
from jax.experimental.pallas import tpu as pltpu
from jax.experimental.pallas import tpu_sc as plsc

B = 16384
C = 128
T = 1000
EPS = 0.0001

_NC = 2
_NS = 16
_NW = _NC * _NS
_BPW = B // _NW
_TPS = 64
_L = 16
_NG = _TPS // _L

_COS_COEFFS = (1.0, -19.73920440673828, 64.93910217285156,
               -85.44971466064453, 60.163578033447266,
               -25.950340270996094, 6.501813888549805)
_INV2PI = np.float32(1.0 / (2.0 * np.pi))
_RB = np.float32(1.5 * 2.0 ** 23)
_SQRT2 = np.float32(np.sqrt(2.0))
_EPS_SCALED = np.float32(np.sqrt(C) * EPS)


def _lane(v, j):
    return jnp.take_along_axis(v, jnp.full((_L,), j, jnp.int32), axis=0)


def _cos_turns(rf):
    k = (rf + _RB) - _RB
    t = rf - k
    s = t * t
    p = jnp.full((_L,), _COS_COEFFS[-1], jnp.float32)
    for c in _COS_COEFFS[-2::-1]:
        p = p * s + np.float32(c)
    return p


def _sc_body(t_hbm, loss_hbm, ab_hbm, lam_hbm, iw_hbm, fq_hbm, ph_hbm,
             w_hbm, mean_hbm, std_hbm, out_hbm, scaled_hbm,
             idx_v, loss_v, f1g_v, f2g_v, o1_v, o2_v, fq_v, ph_v, w_v,
             ms_v, ab_v, lam_v, iw_v, f1loc, f2loc, f1_sh, f2_sh,
             *sems):
    cid = lax.axis_index("c")
    sid = lax.axis_index("s")
    wid = sid * _NC + cid
    base = wid * _BPW
    tbase = jnp.minimum(sid * _TPS, T - _TPS)

    copies = [
        pltpu.async_copy(fq_hbm, fq_v, sems[0]),
        pltpu.async_copy(ph_hbm, ph_v, sems[1]),
        pltpu.async_copy(w_hbm, w_v, sems[2]),
        pltpu.async_copy(mean_hbm, ms_v.at[pl.ds(0, 1)], sems[3]),
        pltpu.async_copy(std_hbm, ms_v.at[pl.ds(8, 1)], sems[4]),
        pltpu.async_copy(ab_hbm.at[pl.ds(tbase, _TPS)], ab_v, sems[5]),
        pltpu.async_copy(lam_hbm.at[pl.ds(tbase, _TPS)], lam_v, sems[6]),
        pltpu.async_copy(iw_hbm.at[pl.ds(tbase, _TPS)], iw_v, sems[7]),
        pltpu.async_copy(t_hbm.at[pl.ds(base, _BPW)], idx_v, sems[8]),
        pltpu.async_copy(loss_hbm.at[pl.ds(base, _BPW)], loss_v, sems[9]),
    ]
    copies[2].wait()

    acc = jnp.zeros((_L,), jnp.float32)
    for v in range(C // _L):
        wv = w_v[pl.ds(v * _L, _L)]
        acc = acc + wv * wv
    svec = jnp.zeros((_L,), jnp.float32) + jnp.sum(acc)
    seed = jnp.full((_L,), np.int32(0x5F3759DF), jnp.int32) - \
        lax.shift_right_logical(plsc.bitcast(svec, jnp.int32), 1)
    y = plsc.bitcast(seed, jnp.float32)
    for _ in range(3):
        y = y * (np.float32(1.5) - (np.float32(0.5) * svec) * y * y)
    normv = svec * y
    scalev = _SQRT2 / (_EPS_SCALED + normv)

    copies[3].wait()
    copies[4].wait()
    copies[5].wait()
    msv = ms_v[pl.ds(0, _L)]
    meanv = _lane(msv, 0)
    stdv = _lane(msv, 8)
    cns = [(ab_v[pl.ds(g * _L, _L)] - meanv) / stdv for g in range(_NG)]

    copies[0].wait()
    copies[1].wait()

    def cbody(c16, accs):
        accs = list(accs)
        coff = c16 * _L
        ft = fq_v[pl.ds(coff, _L)] * _INV2PI
        pt = ph_v[pl.ds(coff, _L)] * _INV2PI
        w16 = w_v[pl.ds(coff, _L)]
        for j in range(_L):
            fb = _lane(ft, j)
            pb = _lane(pt, j)
            wb = _lane(w16, j)
            for g in range(_NG):
                accs[g] = accs[g] + _cos_turns(cns[g] * fb + pb) * wb
        return tuple(accs)

    zero = jnp.zeros((_L,), jnp.float32)
    accs = lax.fori_loop(0, C // _L, cbody, (zero,) * _NG)

    copies[6].wait()
    copies[7].wait()
    for g in range(_NG):
        alw = accs[g] * scalev
        sl = pl.ds(g * _L, _L)
        f1loc[sl] = lam_v[sl] * jnp.exp(-alw)
        f2loc[sl] = iw_v[sl] * alw

    pltpu.sync_copy(f1loc, f1_sh.at[pl.ds(tbase, _TPS)])
    pltpu.sync_copy(f2loc, f2_sh.at[pl.ds(tbase, _TPS)])
    copies[8].wait()
    plsc.subcore_barrier()

    gathers = [
        pltpu.async_copy(f1_sh.at[idx_v], f1g_v, sems[10]),
        pltpu.async_copy(f2_sh.at[idx_v], f2g_v, sems[11]),
    ]
    copies[9].wait()
    for g_ in gathers:
        g_.wait()
    for i in range(_BPW // _L):
        sl = pl.ds(i * _L, _L)
        ls = loss_v[sl] * f1g_v[sl]
        o2_v[sl] = ls
        o1_v[sl] = ls + f2g_v[sl]
    pltpu.sync_copy(o1_v, out_hbm.at[pl.ds(base, _BPW)])
    pltpu.sync_copy(o2_v, scaled_hbm.at[pl.ds(base, _BPW)])


@functools.cache
def _get_sc_kernel():
    return pl.kernel(
        _sc_body,
        out_type=(jax.ShapeDtypeStruct((B,), jnp.float32),
                  jax.ShapeDtypeStruct((B,), jnp.float32)),
        mesh=plsc.VectorSubcoreMesh(core_axis_name="c",
                                    subcore_axis_name="s",
                                    num_cores=_NC, num_subcores=_NS),
        compiler_params=pltpu.CompilerParams(needs_layout_passes=False),
        scratch_types=[
            pltpu.VMEM((_BPW,), jnp.int32),
            pltpu.VMEM((_BPW,), jnp.float32),
            pltpu.VMEM((_BPW,), jnp.float32),
            pltpu.VMEM((_BPW,), jnp.float32),
            pltpu.VMEM((_BPW,), jnp.float32),
            pltpu.VMEM((_BPW,), jnp.float32),
            pltpu.VMEM((C,), jnp.float32),
            pltpu.VMEM((C,), jnp.float32),
            pltpu.VMEM((C,), jnp.float32),
            pltpu.VMEM((_L,), jnp.float32),
            pltpu.VMEM((_TPS,), jnp.float32),
            pltpu.VMEM((_TPS,), jnp.float32),
            pltpu.VMEM((_TPS,), jnp.float32),
            pltpu.VMEM((_TPS,), jnp.float32),
            pltpu.VMEM((_TPS,), jnp.float32),
            pltpu.VMEM_SHARED((T,), jnp.float32),
            pltpu.VMEM_SHARED((T,), jnp.float32),
        ] + [pltpu.SemaphoreType.DMA] * 18,
    )


def kernel(loss, timesteps, freqs, phases, weight, alphas_cumprod,
           a_bar_mean, a_bar_std, lambda_weights, importance_weights):
    loss_out, loss_scaled = _get_sc_kernel()(
        timesteps.astype(jnp.int32), loss, alphas_cumprod, lambda_weights,
        importance_weights, freqs, phases, weight.reshape(C),
        a_bar_mean.reshape(1), a_bar_std.reshape(1))
    return (loss_out, loss_scaled)


# speedup vs baseline: 1.3481x; 1.0257x over previous
import functools

import jax
import jax.numpy as jnp
import numpy as np
from jax import lax
from jax.experimental import pallas as pl
from jax.experimental.pallas import tpu as pltpu
from jax.experimental.pallas import tpu_sc as plsc

B = 16384
C = 128
T = 1000
EPS = 0.0001

_NC = 2
_NS = 16
_NW = _NC * _NS
_BPW = B // _NW
_TPS = 64
_L = 16
_NG = _TPS // _L

_COS_COEFFS = (0.9999616146087646, -19.73118019104004, 64.67359161376953,
               -82.38470458984375, 45.564125061035156)
_INV2PI = np.float32(1.0 / (2.0 * np.pi))
_RB = np.float32(1.5 * 2.0 ** 23)
_SQRT2 = np.float32(np.sqrt(2.0))
_EPS_SCALED = np.float32(np.sqrt(C) * EPS)


def _lane(v, j):
    return jnp.take_along_axis(v, jnp.full((_L,), j, jnp.int32), axis=0)


def _cos_turns(rf):
    k = (rf + _RB) - _RB
    t = rf - k
    s = t * t
    p = jnp.full((_L,), _COS_COEFFS[-1], jnp.float32)
    for c in _COS_COEFFS[-2::-1]:
        p = p * s + np.float32(c)
    return p


def _sc_body(t_hbm, loss_hbm, ab_hbm, lam_hbm, iw_hbm, fq_hbm, ph_hbm,
             w_hbm, mean_hbm, std_hbm, out_hbm, scaled_hbm,
             idx_v, loss_v, f1g_v, f2g_v, o1_v, o2_v, fq_v, ph_v, w_v,
             ms_v, ab_v, lam_v, iw_v, f1loc, f2loc, f1_sh, f2_sh,
             *sems):
    cid = lax.axis_index("c")
    sid = lax.axis_index("s")
    wid = sid * _NC + cid
    base = wid * _BPW
    tbase = jnp.minimum(sid * _TPS, T - _TPS)

    copies = [
        pltpu.async_copy(fq_hbm, fq_v, sems[0]),
        pltpu.async_copy(ph_hbm, ph_v, sems[1]),
        pltpu.async_copy(w_hbm, w_v, sems[2]),
        pltpu.async_copy(mean_hbm, ms_v.at[pl.ds(0, 1)], sems[3]),
        pltpu.async_copy(std_hbm, ms_v.at[pl.ds(8, 1)], sems[4]),
        pltpu.async_copy(ab_hbm.at[pl.ds(tbase, _TPS)], ab_v, sems[5]),
        pltpu.async_copy(lam_hbm.at[pl.ds(tbase, _TPS)], lam_v, sems[6]),
        pltpu.async_copy(iw_hbm.at[pl.ds(tbase, _TPS)], iw_v, sems[7]),
        pltpu.async_copy(t_hbm.at[pl.ds(base, _BPW)], idx_v, sems[8]),
        pltpu.async_copy(loss_hbm.at[pl.ds(base, _BPW)], loss_v, sems[9]),
    ]
    copies[2].wait()

    acc = jnp.zeros((_L,), jnp.float32)
    for v in range(C // _L):
        wv = w_v[pl.ds(v * _L, _L)]
        acc = acc + wv * wv
    svec = jnp.zeros((_L,), jnp.float32) + jnp.sum(acc)
    seed = jnp.full((_L,), np.int32(0x5F3759DF), jnp.int32) - \
        lax.shift_right_logical(plsc.bitcast(svec, jnp.int32), 1)
    y = plsc.bitcast(seed, jnp.float32)
    for _ in range(3):
        y = y * (np.float32(1.5) - (np.float32(0.5) * svec) * y * y)
    normv = svec * y
    scalev = _SQRT2 / (_EPS_SCALED + normv)

    copies[3].wait()
    copies[4].wait()
    copies[5].wait()
    msv = ms_v[pl.ds(0, _L)]
    meanv = _lane(msv, 0)
    stdv = _lane(msv, 8)
    cns = [(ab_v[pl.ds(g * _L, _L)] - meanv) / stdv for g in range(_NG)]

    copies[0].wait()
    copies[1].wait()

    def cbody(c16, accs):
        accs = list(accs)
        coff = c16 * _L
        ft = fq_v[pl.ds(coff, _L)] * _INV2PI
        pt = ph_v[pl.ds(coff, _L)] * _INV2PI
        w16 = w_v[pl.ds(coff, _L)]
        for j in range(_L):
            fb = _lane(ft, j)
            pb = _lane(pt, j)
            wb = _lane(w16, j)
            for g in range(_NG):
                accs[g] = accs[g] + _cos_turns(cns[g] * fb + pb) * wb
        return tuple(accs)

    zero = jnp.zeros((_L,), jnp.float32)
    accs = lax.fori_loop(0, C // _L, cbody, (zero,) * _NG)

    copies[6].wait()
    copies[7].wait()
    for g in range(_NG):
        alw = accs[g] * scalev
        sl = pl.ds(g * _L, _L)
        f1loc[sl] = lam_v[sl] * jnp.exp(-alw)
        f2loc[sl] = iw_v[sl] * alw

    pltpu.sync_copy(f1loc, f1_sh.at[pl.ds(tbase, _TPS)])
    pltpu.sync_copy(f2loc, f2_sh.at[pl.ds(tbase, _TPS)])
    copies[8].wait()
    plsc.subcore_barrier()

    gathers = [
        pltpu.async_copy(f1_sh.at[idx_v], f1g_v, sems[10]),
        pltpu.async_copy(f2_sh.at[idx_v], f2g_v, sems[11]),
    ]
    copies[9].wait()
    for g_ in gathers:
        g_.wait()
    for i in range(_BPW // _L):
        sl = pl.ds(i * _L, _L)
        ls = loss_v[sl] * f1g_v[sl]
        o2_v[sl] = ls
        o1_v[sl] = ls + f2g_v[sl]
    pltpu.sync_copy(o1_v, out_hbm.at[pl.ds(base, _BPW)])
    pltpu.sync_copy(o2_v, scaled_hbm.at[pl.ds(base, _BPW)])


@functools.cache
def _get_sc_kernel():
    return pl.kernel(
        _sc_body,
        out_type=(jax.ShapeDtypeStruct((B,), jnp.float32),
                  jax.ShapeDtypeStruct((B,), jnp.float32)),
        mesh=plsc.VectorSubcoreMesh(core_axis_name="c",
                                    subcore_axis_name="s",
                                    num_cores=_NC, num_subcores=_NS),
        compiler_params=pltpu.CompilerParams(needs_layout_passes=False),
        scratch_types=[
            pltpu.VMEM((_BPW,), jnp.int32),
            pltpu.VMEM((_BPW,), jnp.float32),
            pltpu.VMEM((_BPW,), jnp.float32),
            pltpu.VMEM((_BPW,), jnp.float32),
            pltpu.VMEM((_BPW,), jnp.float32),
            pltpu.VMEM((_BPW,), jnp.float32),
            pltpu.VMEM((C,), jnp.float32),
            pltpu.VMEM((C,), jnp.float32),
            pltpu.VMEM((C,), jnp.float32),
            pltpu.VMEM((_L,), jnp.float32),
            pltpu.VMEM((_TPS,), jnp.float32),
            pltpu.VMEM((_TPS,), jnp.float32),
            pltpu.VMEM((_TPS,), jnp.float32),
            pltpu.VMEM((_TPS,), jnp.float32),
            pltpu.VMEM((_TPS,), jnp.float32),
            pltpu.VMEM_SHARED((T,), jnp.float32),
            pltpu.VMEM_SHARED((T,), jnp.float32),
        ] + [pltpu.SemaphoreType.DMA] * 18,
    )


def kernel(loss, timesteps, freqs, phases, weight, alphas_cumprod,
           a_bar_mean, a_bar_std, lambda_weights, importance_weights):
    loss_out, loss_scaled = _get_sc_kernel()(
        timesteps.astype(jnp.int32), loss, alphas_cumprod, lambda_weights,
        importance_weights, freqs, phases, weight.reshape(C),
        a_bar_mean.reshape(1), a_bar_std.reshape(1))
    return (loss_out, loss_scaled)


# speedup vs baseline: 1.3484x; 1.0003x over previous
import functools

import jax
import jax.numpy as jnp
import numpy as np
from jax import lax
from jax.experimental import pallas as pl
from jax.experimental.pallas import tpu as pltpu
from jax.experimental.pallas import tpu_sc as plsc

B = 16384
C = 128
T = 1000
EPS = 0.0001

_NC = 2
_NS = 16
_NW = _NC * _NS
_BPW = B // _NW
_TPS = 64
_L = 16
_NG = _TPS // _L

_COS_COEFFS = (0.9999616146087646, -19.73118019104004, 64.67359161376953,
               -82.38470458984375, 45.564125061035156)
_INV2PI = np.float32(1.0 / (2.0 * np.pi))
_RB = np.float32(1.5 * 2.0 ** 23)
_SQRT2 = np.float32(np.sqrt(2.0))
_EPS_SCALED = np.float32(np.sqrt(C) * EPS)


def _lane(v, j):
    return jnp.take_along_axis(v, jnp.full((_L,), j, jnp.int32), axis=0)


def _cos_turns(rf):
    k = (rf + _RB) - _RB
    t = rf - k
    s = t * t
    p = jnp.full((_L,), _COS_COEFFS[-1], jnp.float32)
    for c in _COS_COEFFS[-2::-1]:
        p = p * s + np.float32(c)
    return p


def _sc_body(t_hbm, loss_hbm, ab_hbm, lam_hbm, iw_hbm, fq_hbm, ph_hbm,
             w_hbm, mean_hbm, std_hbm, out_hbm, scaled_hbm,
             idx_v, loss_v, f1g_v, f2g_v, o1_v, o2_v, fq_v, ph_v, w_v,
             ms_v, ab_v, lam_v, iw_v, f1loc, f2loc, f1_sh, f2_sh,
             *sems):
    cid = lax.axis_index("c")
    sid = lax.axis_index("s")
    wid = sid * _NC + cid
    base = wid * _BPW
    tbase = jnp.minimum(sid * _TPS, T - _TPS)

    copies = [
        pltpu.async_copy(fq_hbm, fq_v, sems[0]),
        pltpu.async_copy(ph_hbm, ph_v, sems[1]),
        pltpu.async_copy(w_hbm, w_v, sems[2]),
        pltpu.async_copy(mean_hbm, ms_v.at[pl.ds(0, 1)], sems[3]),
        pltpu.async_copy(std_hbm, ms_v.at[pl.ds(8, 1)], sems[4]),
        pltpu.async_copy(ab_hbm.at[pl.ds(tbase, _TPS)], ab_v, sems[5]),
        pltpu.async_copy(lam_hbm.at[pl.ds(tbase, _TPS)], lam_v, sems[6]),
        pltpu.async_copy(iw_hbm.at[pl.ds(tbase, _TPS)], iw_v, sems[7]),
        pltpu.async_copy(t_hbm.at[pl.ds(base, _BPW)], idx_v, sems[8]),
        pltpu.async_copy(loss_hbm.at[pl.ds(base, _BPW)], loss_v, sems[9]),
    ]
    copies[2].wait()

    acc = jnp.zeros((_L,), jnp.float32)
    for v in range(C // _L):
        wv = w_v[pl.ds(v * _L, _L)]
        acc = acc + wv * wv
    svec = jnp.zeros((_L,), jnp.float32) + jnp.sum(acc)
    seed = jnp.full((_L,), np.int32(0x5F3759DF), jnp.int32) - \
        lax.shift_right_logical(plsc.bitcast(svec, jnp.int32), 1)
    y = plsc.bitcast(seed, jnp.float32)
    for _ in range(3):
        y = y * (np.float32(1.5) - (np.float32(0.5) * svec) * y * y)
    normv = svec * y
    scalev = _SQRT2 / (_EPS_SCALED + normv)

    copies[3].wait()
    copies[4].wait()
    copies[5].wait()
    msv = ms_v[pl.ds(0, _L)]
    meanv = _lane(msv, 0)
    stdv = _lane(msv, 8)
    cns = [(ab_v[pl.ds(g * _L, _L)] - meanv) / stdv for g in range(_NG)]

    copies[0].wait()
    copies[1].wait()

    def cbody(c16, accs):
        accs = list(accs)
        coff = c16 * _L
        ft = fq_v[pl.ds(coff, _L)] * _INV2PI
        pt = ph_v[pl.ds(coff, _L)] * _INV2PI
        w16 = w_v[pl.ds(coff, _L)]
        for j in range(_L):
            fb = _lane(ft, j)
            pb = _lane(pt, j)
            wb = _lane(w16, j)
            for g in range(_NG):
                accs[g] = accs[g] + _cos_turns(cns[g] * fb + pb) * wb
        return tuple(accs)

    zero = jnp.zeros((_L,), jnp.float32)
    accs = lax.fori_loop(0, C // _L, cbody, (zero,) * _NG)

    copies[6].wait()
    copies[7].wait()
    for g in range(_NG):
        alw = accs[g] * scalev
        sl = pl.ds(g * _L, _L)
        f1loc[sl] = lam_v[sl] * jnp.exp(-alw)
        f2loc[sl] = iw_v[sl] * alw

    pltpu.sync_copy(f1loc, f1_sh.at[pl.ds(tbase, _TPS)])
    pltpu.sync_copy(f2loc, f2_sh.at[pl.ds(tbase, _TPS)])
    copies[8].wait()
    plsc.subcore_barrier()

    gathers = [
        pltpu.async_copy(f1_sh.at[idx_v], f1g_v, sems[10]),
        pltpu.async_copy(f2_sh.at[idx_v], f2g_v, sems[11]),
    ]
    copies[9].wait()
    for g_ in gathers:
        g_.wait()
    half = _BPW // 2
    outs = []
    for h in range(2):
        for i in range(half // _L):
            sl = pl.ds(h * half + i * _L, _L)
            ls = loss_v[sl] * f1g_v[sl]
            o2_v[sl] = ls
            o1_v[sl] = ls + f2g_v[sl]
        hs = pl.ds(h * half, half)
        outs.append(pltpu.async_copy(
            o1_v.at[hs], out_hbm.at[pl.ds(base + h * half, half)],
            sems[12 + h]))
        outs.append(pltpu.async_copy(
            o2_v.at[hs], scaled_hbm.at[pl.ds(base + h * half, half)],
            sems[14 + h]))
    for o_ in outs:
        o_.wait()


@functools.cache
def _get_sc_kernel():
    return pl.kernel(
        _sc_body,
        out_type=(jax.ShapeDtypeStruct((B,), jnp.float32),
                  jax.ShapeDtypeStruct((B,), jnp.float32)),
        mesh=plsc.VectorSubcoreMesh(core_axis_name="c",
                                    subcore_axis_name="s",
                                    num_cores=_NC, num_subcores=_NS),
        compiler_params=pltpu.CompilerParams(needs_layout_passes=False),
        scratch_types=[
            pltpu.VMEM((_BPW,), jnp.int32),
            pltpu.VMEM((_BPW,), jnp.float32),
            pltpu.VMEM((_BPW,), jnp.float32),
            pltpu.VMEM((_BPW,), jnp.float32),
            pltpu.VMEM((_BPW,), jnp.float32),
            pltpu.VMEM((_BPW,), jnp.float32),
            pltpu.VMEM((C,), jnp.float32),
            pltpu.VMEM((C,), jnp.float32),
            pltpu.VMEM((C,), jnp.float32),
            pltpu.VMEM((_L,), jnp.float32),
            pltpu.VMEM((_TPS,), jnp.float32),
            pltpu.VMEM((_TPS,), jnp.float32),
            pltpu.VMEM((_TPS,), jnp.float32),
            pltpu.VMEM((_TPS,), jnp.float32),
            pltpu.VMEM((_TPS,), jnp.float32),
            pltpu.VMEM_SHARED((T,), jnp.float32),
            pltpu.VMEM_SHARED((T,), jnp.float32),
        ] + [pltpu.SemaphoreType.DMA] * 18,
    )


def kernel(loss, timesteps, freqs, phases, weight, alphas_cumprod,
           a_bar_mean, a_bar_std, lambda_weights, importance_weights):
    loss_out, loss_scaled = _get_sc_kernel()(
        timesteps.astype(jnp.int32), loss, alphas_cumprod, lambda_weights,
        importance_weights, freqs, phases, weight.reshape(C),
        a_bar_mean.reshape(1), a_bar_std.reshape(1))
    return (loss_out, loss_scaled)
